# baseline (speedup 1.0000x reference)
import functools

import jax
import jax.numpy as jnp
from jax import lax
from jax.experimental import pallas as pl
from jax.experimental.pallas import tpu as pltpu
from jax.experimental.pallas import tpu_sc as plsc

D = 128
LANES = 16
NC, NS = 2, 16
NW = NC * NS
E_TOTAL = 320000
E_PER_W = E_TOTAL // NW
CHUNK = 16
NCHUNK = E_PER_W // CHUNK
NSTREAM = 3
NBODY = (NCHUNK - 1) // NSTREAM

_GATHER_DN = lax.GatherDimensionNumbers(
    offset_dims=(), collapsed_slice_dims=(0,), start_index_map=(0,))


def _lane_perm(x, idx):
    return lax.gather(x, idx[:, None], _GATHER_DN, slice_sizes=(1,),
                      mode=lax.GatherScatterMode.PROMISE_IN_BOUNDS)


def _dot_decoder_sc(z, src, dst):
    mesh = plsc.VectorSubcoreMesh(core_axis_name="c", subcore_axis_name="s")

    @functools.partial(
        pl.kernel,
        mesh=mesh,
        out_type=jax.ShapeDtypeStruct((E_TOTAL,), jnp.float32),
        scratch_types=[
            pltpu.VMEM((E_PER_W,), jnp.int32),
            pltpu.VMEM((E_PER_W,), jnp.int32),
            pltpu.VMEM((CHUNK, D), jnp.float32),
            pltpu.VMEM((CHUNK, D), jnp.float32),
            pltpu.VMEM((CHUNK, D), jnp.float32),
            pltpu.VMEM((CHUNK, D), jnp.float32),
            pltpu.VMEM((CHUNK, D), jnp.float32),
            pltpu.VMEM((CHUNK, D), jnp.float32),
            pltpu.VMEM_SHARED((10000, D), jnp.float32),
            pltpu.VMEM((CHUNK,), jnp.float32),
            pltpu.VMEM((CHUNK,), jnp.float32),
            pltpu.VMEM((CHUNK,), jnp.float32),
            pltpu.SemaphoreType.DMA,
            pltpu.SemaphoreType.DMA,
            pltpu.SemaphoreType.DMA,
            pltpu.SemaphoreType.DMA,
            pltpu.SemaphoreType.DMA,
            pltpu.SemaphoreType.DMA,
        ],
    )
    def k(z_hbm, src_hbm, dst_hbm, out_hbm,
          sidx, didx, sr_a0, dr_a0, sr_a1, dr_a1, sr_b, dr_b, zsh,
          out_a0, out_a1, out_b,
          sem_a0, sem_a1, sem_b, sem_oa0, sem_oa1, sem_ob):
        sid = lax.axis_index("s")
        wid = sid * NC + lax.axis_index("c")
        base = wid * E_PER_W

        @pl.when(sid == 0)
        def _():
            pltpu.sync_copy(z_hbm, zsh)

        pltpu.sync_copy(src_hbm.at[pl.ds(base, E_PER_W)], sidx)
        pltpu.sync_copy(dst_hbm.at[pl.ds(base, E_PER_W)], didx)
        plsc.subcore_barrier()

        lane = lax.iota(jnp.int32, 16)

        def fire(j, table, srows, drows, sem):
            c0 = j * CHUNK
            pltpu.async_copy(table.at[sidx.at[pl.ds(c0, CHUNK)]], srows, sem)
            pltpu.async_copy(table.at[didx.at[pl.ds(c0, CHUNK)]], drows, sem)

        def drain(srows, drows, sem):
            pltpu.make_async_copy(zsh.at[sidx.at[pl.ds(0, CHUNK)]],
                                  srows, sem).wait()
            pltpu.make_async_copy(zsh.at[didx.at[pl.ds(0, CHUNK)]],
                                  drows, sem).wait()

        def compute(j, srows, drows, outb, sem_o):
            c0 = j * CHUNK

            @pl.when(j >= NSTREAM)
            def _():
                pltpu.make_async_copy(
                    outb, out_hbm.at[pl.ds(base, CHUNK)], sem_o).wait()

            out16 = jnp.zeros((LANES,), jnp.float32)
            for i in range(LANES):
                acc = jnp.zeros((LANES,), jnp.float32)
                for f in range(D // LANES):
                    acc = acc + (srows[i, pl.ds(f * LANES, LANES)]
                                 * drows[i, pl.ds(f * LANES, LANES)])
                for sh in (8, 4, 2, 1):
                    acc = acc + _lane_perm(acc, lane ^ sh)
                out16 = jnp.where(lane == i, acc, out16)
            outb[pl.ds(0, LANES)] = out16
            pltpu.async_copy(outb, out_hbm.at[pl.ds(base + c0, CHUNK)], sem_o)

        fire(0, zsh, sr_a0, dr_a0, sem_a0)
        fire(1, zsh, sr_a1, dr_a1, sem_a1)
        fire(2, z_hbm, sr_b, dr_b, sem_b)

        def body(p, _):
            j = p * NSTREAM

            drain(sr_a0, dr_a0, sem_a0)
            compute(j, sr_a0, dr_a0, out_a0, sem_oa0)
            fire(j + 3, zsh, sr_a0, dr_a0, sem_a0)

            drain(sr_a1, dr_a1, sem_a1)
            compute(j + 1, sr_a1, dr_a1, out_a1, sem_oa1)

            @pl.when(j + 4 < NCHUNK)
            def _():
                fire(j + 4, zsh, sr_a1, dr_a1, sem_a1)

            drain(sr_b, dr_b, sem_b)
            compute(j + 2, sr_b, dr_b, out_b, sem_ob)

            @pl.when(j + 5 < NCHUNK)
            def _():
                fire(j + 5, z_hbm, sr_b, dr_b, sem_b)

            return ()

        lax.fori_loop(0, NBODY, body, (), unroll=False)

        drain(sr_a0, dr_a0, sem_a0)
        compute(NCHUNK - 1, sr_a0, dr_a0, out_a0, sem_oa0)

        for outb, sem_o in ((out_a0, sem_oa0), (out_a1, sem_oa1),
                            (out_b, sem_ob)):
            pltpu.make_async_copy(outb, out_hbm.at[pl.ds(base, CHUNK)],
                                  sem_o).wait()

    return k(z, src, dst)


def kernel(z, edge_index):
    src = edge_index[0].astype(jnp.int32)
    dst = edge_index[1].astype(jnp.int32)
    return _dot_decoder_sc(z, src, dst)


# speedup vs baseline: 1.0716x; 1.0716x over previous
import functools

import jax
import jax.numpy as jnp
from jax import lax
from jax.experimental import pallas as pl
from jax.experimental.pallas import tpu as pltpu
from jax.experimental.pallas import tpu_sc as plsc

D = 128
LANES = 16
NC, NS = 2, 16
NW = NC * NS
E_TOTAL = 320000
E_PER_W = E_TOTAL // NW
CHUNK = 16
NCHUNK = E_PER_W // CHUNK
NSTREAM = 3
NBODY = (NCHUNK - 1) // NSTREAM

_GATHER_DN = lax.GatherDimensionNumbers(
    offset_dims=(), collapsed_slice_dims=(0,), start_index_map=(0,))


def _lane_perm(x, idx):
    return lax.gather(x, idx[:, None], _GATHER_DN, slice_sizes=(1,),
                      mode=lax.GatherScatterMode.PROMISE_IN_BOUNDS)


def _dot_decoder_sc(z, src, dst):
    mesh = plsc.VectorSubcoreMesh(core_axis_name="c", subcore_axis_name="s")

    @functools.partial(
        pl.kernel,
        mesh=mesh,
        out_type=jax.ShapeDtypeStruct((E_TOTAL,), jnp.float32),
        scratch_types=[
            pltpu.VMEM((E_PER_W,), jnp.int32),
            pltpu.VMEM((E_PER_W,), jnp.int32),
            pltpu.VMEM((CHUNK, D), jnp.float32),
            pltpu.VMEM((CHUNK, D), jnp.float32),
            pltpu.VMEM((CHUNK, D), jnp.float32),
            pltpu.VMEM((CHUNK, D), jnp.float32),
            pltpu.VMEM((CHUNK, D), jnp.float32),
            pltpu.VMEM((CHUNK, D), jnp.float32),
            pltpu.VMEM_SHARED((10000, D), jnp.float32),
            pltpu.VMEM((CHUNK,), jnp.float32),
            pltpu.VMEM((CHUNK,), jnp.float32),
            pltpu.VMEM((CHUNK,), jnp.float32),
            pltpu.SemaphoreType.DMA,
            pltpu.SemaphoreType.DMA,
            pltpu.SemaphoreType.DMA,
            pltpu.SemaphoreType.DMA,
            pltpu.SemaphoreType.DMA,
            pltpu.SemaphoreType.DMA,
        ],
    )
    def k(z_hbm, src_hbm, dst_hbm, out_hbm,
          sidx, didx, sr_a0, dr_a0, sr_a1, dr_a1, sr_b, dr_b, zsh,
          out_a0, out_a1, out_b,
          sem_a0, sem_a1, sem_b, sem_oa0, sem_oa1, sem_ob):
        sid = lax.axis_index("s")
        wid = sid * NC + lax.axis_index("c")
        base = wid * E_PER_W

        @pl.when(sid == 0)
        def _():
            pltpu.sync_copy(z_hbm, zsh)

        pltpu.sync_copy(src_hbm.at[pl.ds(base, E_PER_W)], sidx)
        pltpu.sync_copy(dst_hbm.at[pl.ds(base, E_PER_W)], didx)
        plsc.subcore_barrier()

        lane = lax.iota(jnp.int32, 16)

        def fire(j, table, srows, drows, sem):
            c0 = j * CHUNK
            pltpu.async_copy(table.at[sidx.at[pl.ds(c0, CHUNK)]], srows, sem)
            pltpu.async_copy(table.at[didx.at[pl.ds(c0, CHUNK)]], drows, sem)

        def drain(srows, drows, sem):
            pltpu.make_async_copy(zsh.at[sidx.at[pl.ds(0, CHUNK)]],
                                  srows, sem).wait()
            pltpu.make_async_copy(zsh.at[didx.at[pl.ds(0, CHUNK)]],
                                  drows, sem).wait()

        def compute(j, srows, drows, outb, sem_o):
            c0 = j * CHUNK

            @pl.when(j >= NSTREAM)
            def _():
                pltpu.make_async_copy(
                    outb, out_hbm.at[pl.ds(base, CHUNK)], sem_o).wait()

            out16 = jnp.zeros((LANES,), jnp.float32)
            for i in range(LANES):
                acc = jnp.zeros((LANES,), jnp.float32)
                for f in range(D // LANES):
                    acc = acc + (srows[i, pl.ds(f * LANES, LANES)]
                                 * drows[i, pl.ds(f * LANES, LANES)])
                for sh in (8, 4, 2, 1):
                    acc = acc + _lane_perm(acc, lane ^ sh)
                out16 = jnp.where(lane == i, acc, out16)
            outb[pl.ds(0, LANES)] = out16
            pltpu.async_copy(outb, out_hbm.at[pl.ds(base + c0, CHUNK)], sem_o)

        fire(0, zsh, sr_a0, dr_a0, sem_a0)
        fire(1, zsh, sr_a1, dr_a1, sem_a1)
        fire(2, zsh, sr_b, dr_b, sem_b)

        def body(p, _):
            j = p * NSTREAM

            drain(sr_a0, dr_a0, sem_a0)
            compute(j, sr_a0, dr_a0, out_a0, sem_oa0)
            fire(j + 3, zsh, sr_a0, dr_a0, sem_a0)

            drain(sr_a1, dr_a1, sem_a1)
            compute(j + 1, sr_a1, dr_a1, out_a1, sem_oa1)

            @pl.when(j + 4 < NCHUNK)
            def _():
                fire(j + 4, zsh, sr_a1, dr_a1, sem_a1)

            drain(sr_b, dr_b, sem_b)
            compute(j + 2, sr_b, dr_b, out_b, sem_ob)

            @pl.when(j + 5 < NCHUNK)
            def _():
                fire(j + 5, zsh, sr_b, dr_b, sem_b)

            return ()

        lax.fori_loop(0, NBODY, body, (), unroll=False)

        drain(sr_a0, dr_a0, sem_a0)
        compute(NCHUNK - 1, sr_a0, dr_a0, out_a0, sem_oa0)

        for outb, sem_o in ((out_a0, sem_oa0), (out_a1, sem_oa1),
                            (out_b, sem_ob)):
            pltpu.make_async_copy(outb, out_hbm.at[pl.ds(base, CHUNK)],
                                  sem_o).wait()

    return k(z, src, dst)


def kernel(z, edge_index):
    src = edge_index[0].astype(jnp.int32)
    dst = edge_index[1].astype(jnp.int32)
    return _dot_decoder_sc(z, src, dst)


# speedup vs baseline: 1.0967x; 1.0235x over previous
import functools

import jax
import jax.numpy as jnp
from jax import lax
from jax.experimental import pallas as pl
from jax.experimental.pallas import tpu as pltpu
from jax.experimental.pallas import tpu_sc as plsc

D = 128
LANES = 16
NC, NS = 2, 16
NW = NC * NS
E_TOTAL = 320000
E_PER_W = E_TOTAL // NW
CHUNK = 16
NCHUNK = E_PER_W // CHUNK

_GATHER_DN = lax.GatherDimensionNumbers(
    offset_dims=(), collapsed_slice_dims=(0,), start_index_map=(0,))


def _lane_perm(x, idx):
    return lax.gather(x, idx[:, None], _GATHER_DN, slice_sizes=(1,),
                      mode=lax.GatherScatterMode.PROMISE_IN_BOUNDS)


def _dot_decoder_sc(z, src, dst):
    mesh = plsc.VectorSubcoreMesh(core_axis_name="c", subcore_axis_name="s")

    @functools.partial(
        pl.kernel,
        mesh=mesh,
        out_type=jax.ShapeDtypeStruct((E_TOTAL,), jnp.float32),
        scratch_types=[
            pltpu.VMEM((E_PER_W,), jnp.int32),
            pltpu.VMEM((E_PER_W,), jnp.int32),
            pltpu.VMEM((CHUNK, D), jnp.float32),
            pltpu.VMEM((CHUNK, D), jnp.float32),
            pltpu.VMEM((CHUNK, D), jnp.float32),
            pltpu.VMEM((CHUNK, D), jnp.float32),
            pltpu.VMEM_SHARED((10000, D), jnp.float32),
            pltpu.VMEM((E_PER_W,), jnp.float32),
            pltpu.SemaphoreType.DMA,
            pltpu.SemaphoreType.DMA,
            pltpu.SemaphoreType.DMA,
            pltpu.SemaphoreType.DMA,
        ],
    )
    def k(z_hbm, src_hbm, dst_hbm, out_hbm,
          sidx, didx, srows_a, drows_a, srows_b, drows_b, zsh, outv,
          sem_sa, sem_da, sem_sb, sem_db):
        sid = lax.axis_index("s")
        wid = sid * NC + lax.axis_index("c")
        base = wid * E_PER_W

        @pl.when(sid == 0)
        def _():
            pltpu.sync_copy(z_hbm, zsh)

        pltpu.sync_copy(src_hbm.at[pl.ds(base, E_PER_W)], sidx)
        pltpu.sync_copy(dst_hbm.at[pl.ds(base, E_PER_W)], didx)
        plsc.subcore_barrier()

        lane = lax.iota(jnp.int32, 16)

        def fire(j, srows, drows, sem_s, sem_d):
            c0 = j * CHUNK
            pltpu.async_copy(zsh.at[sidx.at[pl.ds(c0, CHUNK)]], srows, sem_s)
            pltpu.async_copy(zsh.at[didx.at[pl.ds(c0, CHUNK)]], drows, sem_d)

        def drain(srows, drows, sem_s, sem_d):
            pltpu.make_async_copy(zsh.at[sidx.at[pl.ds(0, CHUNK)]],
                                  srows, sem_s).wait()
            pltpu.make_async_copy(zsh.at[didx.at[pl.ds(0, CHUNK)]],
                                  drows, sem_d).wait()

        masks = [(lane & sh) != 0 for sh in (1, 2, 4, 8)]
        perms = [lane ^ sh for sh in (1, 2, 4, 8)]

        def compute(j, srows, drows):
            c0 = j * CHUNK
            accs = []
            for i in range(LANES):
                acc = jnp.zeros((LANES,), jnp.float32)
                for f in range(D // LANES):
                    acc = acc + (srows[i, pl.ds(f * LANES, LANES)]
                                 * drows[i, pl.ds(f * LANES, LANES)])
                accs.append(acc)
            for m, pidx in zip(masks, perms):
                accs = [jnp.where(m, b, a) + _lane_perm(jnp.where(m, a, b),
                                                        pidx)
                        for a, b in zip(accs[0::2], accs[1::2])]
            outv[pl.ds(c0, LANES)] = accs[0]

        fire(0, srows_a, drows_a, sem_sa, sem_da)

        def pair_body(p, _):
            g = p * 2
            fire(g + 1, srows_b, drows_b, sem_sb, sem_db)
            drain(srows_a, drows_a, sem_sa, sem_da)
            compute(g, srows_a, drows_a)
            fire(g + 2, srows_a, drows_a, sem_sa, sem_da)
            drain(srows_b, drows_b, sem_sb, sem_db)
            compute(g + 1, srows_b, drows_b)
            return ()

        lax.fori_loop(0, (NCHUNK - 1) // 2, pair_body, (), unroll=False)

        drain(srows_a, drows_a, sem_sa, sem_da)
        compute(NCHUNK - 1, srows_a, drows_a)

        pltpu.sync_copy(outv, out_hbm.at[pl.ds(base, E_PER_W)])

    return k(z, src, dst)


def kernel(z, edge_index):
    src = edge_index[0].astype(jnp.int32)
    dst = edge_index[1].astype(jnp.int32)
    return _dot_decoder_sc(z, src, dst)
